# baseline (speedup 1.0000x reference)
import functools

import jax
import jax.numpy as jnp
from jax import lax
from jax.experimental import pallas as pl
from jax.experimental.pallas import tpu as pltpu
from jax.experimental.pallas import tpu_sc as plsc

_B, _C = 8, 128
_HF, _WF = 128, 128
_HC, _WC = 64, 64
_HS, _WS = 512, 512
_NF, _NCB = 8192, 8192


def _sc_gather(table, idx2d):
    info = plsc.get_sparse_core_info()
    n_cores, n_sub = info.num_cores, info.num_subcores
    nw = n_cores * n_sub
    nrows, lanes = idx2d.shape
    rows_per_w = nrows // nw
    d = table.shape[1]

    mesh = plsc.VectorSubcoreMesh(core_axis_name="c", subcore_axis_name="s")

    @functools.partial(
        pl.kernel,
        mesh=mesh,
        out_type=jax.ShapeDtypeStruct((nrows * lanes, d), jnp.float32),
        scratch_types=[
            pltpu.VMEM((rows_per_w, lanes), jnp.int32),
            pltpu.VMEM((lanes, d), jnp.float32),
            pltpu.VMEM((lanes, d), jnp.float32),
            pltpu.SemaphoreType.DMA,
            pltpu.SemaphoreType.DMA,
        ],
    )
    def gather_k(table_hbm, idx_hbm, out_hbm, idx_v, rows_a, rows_b, sem_a, sem_b):
        wid = lax.axis_index("s") * n_cores + lax.axis_index("c")
        base = wid * rows_per_w
        pltpu.sync_copy(idx_hbm.at[pl.ds(base, rows_per_w)], idx_v)
        bufs = (rows_a, rows_b)
        sems = (sem_a, sem_b)
        pltpu.async_copy(table_hbm.at[idx_v.at[0]], rows_a, sem_a)

        def body(t, carry):
            j0 = 2 * t
            for bsel in range(2):
                j = j0 + bsel
                buf, sem = bufs[bsel], sems[bsel]
                nbuf, nsem = bufs[1 - bsel], sems[1 - bsel]
                pltpu.make_async_copy(table_hbm.at[idx_v.at[j]], buf, sem).wait()

                @pl.when(j + 1 < rows_per_w)
                def _prefetch():
                    pltpu.async_copy(table_hbm.at[idx_v.at[j + 1]], nbuf, nsem)

                pltpu.sync_copy(buf, out_hbm.at[pl.ds((base + j) * lanes, lanes)])
            return carry

        lax.fori_loop(0, rows_per_w // 2, body, 0)

    return gather_k(table, idx2d)


def _blend(M, q3, g, h_lat, w_lat, pool):
    hb = 8
    n_hblk = h_lat // hb
    spec_rows = hb * pool
    fl = hb * w_lat

    def body(m_ref, q_ref, g_ref, out_ref):
        m = m_ref[0, 0]
        a_i = lax.broadcasted_iota(jnp.int32, (hb, spec_rows), 0)
        a_j = lax.broadcasted_iota(jnp.int32, (hb, spec_rows), 1)
        amat = (a_j // pool == a_i).astype(jnp.float32)
        p_i = lax.broadcasted_iota(jnp.int32, (_WS, w_lat), 0)
        p_j = lax.broadcasted_iota(jnp.int32, (_WS, w_lat), 1)
        pmat = (p_i // pool == p_j).astype(jnp.float32)
        s = jnp.dot(amat, m, preferred_element_type=jnp.float32)
        pooled = jnp.dot(s, pmat, preferred_element_type=jnp.float32)
        x = jnp.concatenate([pooled] * hb, axis=1)
        u_h = lax.broadcasted_iota(jnp.int32, (hb, fl), 0)
        u_p = lax.broadcasted_iota(jnp.int32, (hb, fl), 1)
        usel = u_p // w_lat == u_h
        maskf = jnp.sum(jnp.where(usel, x, 0.0), axis=0, keepdims=True)
        gt = g_ref[...].T
        out_ref[0, :, :] = jnp.where(maskf > 0.0, gt, q_ref[0, :, :])

    return pl.pallas_call(
        body,
        grid=(_B, n_hblk),
        in_specs=[
            pl.BlockSpec((1, 1, spec_rows, _WS), lambda b, i: (b, 0, i, 0)),
            pl.BlockSpec((1, _C, fl), lambda b, i: (b, 0, i)),
            pl.BlockSpec((fl, _C), lambda b, i: (b * n_hblk + i, 0)),
        ],
        out_specs=pl.BlockSpec((1, _C, fl), lambda b, i: (b, 0, i)),
        out_shape=jax.ShapeDtypeStruct(q3.shape, q3.dtype),
    )(M, q3, g)


def kernel(q_fine, q_coarse, M, codebook_fine, codebook_coarse):
    key = jax.random.key(42)
    kf, kc = jax.random.split(key)
    idx_f = jax.random.randint(kf, (_B, _HF, _WF), 0, _NF)
    idx_c = jax.random.randint(kc, (_B, _HC, _WC), 0, _NCB)

    g_c = _sc_gather(codebook_coarse, idx_c.reshape(-1, 128).astype(jnp.int32))
    g_f = _sc_gather(codebook_fine, idx_f.reshape(-1, 128).astype(jnp.int32))

    q3_f = q_fine.reshape(_B, _C, _HF * _WF)
    q3_c = q_coarse.reshape(_B, _C, _HC * _WC)
    aug_c = _blend(M, q3_c, g_c, _HC, _WC, _HS // _HC)
    aug_f = _blend(M, q3_f, g_f, _HF, _WF, _HS // _HF)
    return (aug_f.reshape(q_fine.shape), aug_c.reshape(q_coarse.shape))


# speedup vs baseline: 1.2198x; 1.2198x over previous
import functools

import jax
import jax.numpy as jnp
from jax import lax
from jax.experimental import pallas as pl
from jax.experimental.pallas import tpu as pltpu
from jax.experimental.pallas import tpu_sc as plsc

_B, _C = 8, 128
_HF, _WF = 128, 128
_HC, _WC = 64, 64
_HS, _WS = 512, 512
_NF, _NCB = 8192, 8192


def _sc_gather(table, idx2d):
    info = plsc.get_sparse_core_info()
    n_cores, n_sub = info.num_cores, info.num_subcores
    nw = n_cores * n_sub
    nrows, lanes = idx2d.shape
    rows_per_w = nrows // nw
    d = table.shape[1]

    mesh = plsc.VectorSubcoreMesh(core_axis_name="c", subcore_axis_name="s")

    @functools.partial(
        pl.kernel,
        mesh=mesh,
        out_type=jax.ShapeDtypeStruct((nrows * lanes, d), jnp.float32),
        scratch_types=[
            pltpu.VMEM((rows_per_w, lanes), jnp.int32),
            pltpu.VMEM((lanes, d), jnp.float32),
            pltpu.VMEM((lanes, d), jnp.float32),
            pltpu.SemaphoreType.DMA,
            pltpu.SemaphoreType.DMA,
        ],
    )
    def gather_k(table_hbm, idx_hbm, out_hbm, idx_v, rows_a, rows_b, sem_a, sem_b):
        wid = lax.axis_index("s") * n_cores + lax.axis_index("c")
        base = wid * rows_per_w
        pltpu.sync_copy(idx_hbm.at[pl.ds(base, rows_per_w)], idx_v)
        bufs = (rows_a, rows_b)
        sems = (sem_a, sem_b)
        pltpu.async_copy(table_hbm.at[idx_v.at[0]], rows_a, sem_a)

        def body(t, carry):
            j0 = 2 * t
            for bsel in range(2):
                j = j0 + bsel
                buf, sem = bufs[bsel], sems[bsel]
                nbuf, nsem = bufs[1 - bsel], sems[1 - bsel]
                pltpu.make_async_copy(table_hbm.at[idx_v.at[j]], buf, sem).wait()

                @pl.when(j + 1 < rows_per_w)
                def _prefetch():
                    pltpu.async_copy(table_hbm.at[idx_v.at[j + 1]], nbuf, nsem)

                pltpu.sync_copy(buf, out_hbm.at[pl.ds((base + j) * lanes, lanes)])
            return carry

        lax.fori_loop(0, rows_per_w // 2, body, 0)

    return gather_k(table, idx2d)


def _blend(M, q, g, h_lat, w_lat, pool):
    hb = 8
    n_hblk = h_lat // hb
    spec_rows = hb * pool

    def body(m_ref, q_ref, g_ref, out_ref):
        m = m_ref[0, 0]
        a_i = lax.broadcasted_iota(jnp.int32, (hb, spec_rows), 0)
        a_j = lax.broadcasted_iota(jnp.int32, (hb, spec_rows), 1)
        amat = (a_j // pool == a_i).astype(jnp.float32)
        p_i = lax.broadcasted_iota(jnp.int32, (_WS, w_lat), 0)
        p_j = lax.broadcasted_iota(jnp.int32, (_WS, w_lat), 1)
        pmat = (p_i // pool == p_j).astype(jnp.float32)
        s = jnp.dot(amat, m, preferred_element_type=jnp.float32)
        pooled = jnp.dot(s, pmat, preferred_element_type=jnp.float32)
        mask3 = (pooled > 0.0)[None]
        gr = g_ref[...].reshape(hb, w_lat, _C)
        gt3 = jnp.transpose(gr, (2, 0, 1))
        out_ref[0] = jnp.where(mask3, gt3, q_ref[0])

    return pl.pallas_call(
        body,
        grid=(_B, n_hblk),
        in_specs=[
            pl.BlockSpec((1, 1, spec_rows, _WS), lambda b, i: (b, 0, i, 0)),
            pl.BlockSpec((1, _C, hb, w_lat), lambda b, i: (b, 0, i, 0)),
            pl.BlockSpec((hb * w_lat, _C), lambda b, i: (b * n_hblk + i, 0)),
        ],
        out_specs=pl.BlockSpec((1, _C, hb, w_lat), lambda b, i: (b, 0, i, 0)),
        out_shape=jax.ShapeDtypeStruct(q.shape, q.dtype),
    )(M, q, g)


def kernel(q_fine, q_coarse, M, codebook_fine, codebook_coarse):
    key = jax.random.key(42)
    kf, kc = jax.random.split(key)
    idx_f = jax.random.randint(kf, (_B, _HF, _WF), 0, _NF)
    idx_c = jax.random.randint(kc, (_B, _HC, _WC), 0, _NCB)

    g_c = _sc_gather(codebook_coarse, idx_c.reshape(-1, 128).astype(jnp.int32))
    g_f = _sc_gather(codebook_fine, idx_f.reshape(-1, 128).astype(jnp.int32))

    aug_c = _blend(M, q_coarse, g_c, _HC, _WC, _HS // _HC)
    aug_f = _blend(M, q_fine, g_f, _HF, _WF, _HS // _HF)
    return (aug_f, aug_c)
